# Initial kernel scaffold; baseline (speedup 1.0000x reference)
import jax
import jax.numpy as jnp
from jax.experimental import pallas as pl


def kernel(x, Wg, W1, b1, W2, b2):
    raise NotImplementedError("write your pallas kernel here")


# speedup vs baseline: 1.8304x; 1.8304x over previous
import functools

import jax
import jax.numpy as jnp
from jax.experimental import pallas as pl
from jax.experimental.pallas import tpu as pltpu

B, S, D, E, FF, TOP_K = 2, 2048, 1024, 8, 4096, 2
N = B * S

_SQRT_HALF = 0.7071067811865476


def _erf(x):
    a1, a2, a3, a4, a5 = (0.254829592, -0.284496736, 1.421413741,
                          -1.453152027, 1.061405429)
    p = 0.3275911
    s = jnp.sign(x)
    ax = jnp.abs(x)
    t = 1.0 / (1.0 + p * ax)
    poly = t * (a1 + t * (a2 + t * (a3 + t * (a4 + t * a5))))
    y = 1.0 - poly * jnp.exp(-ax * ax)
    return s * y


def _gelu(x):
    return 0.5 * x * (1.0 + _erf(x * _SQRT_HALF))


def _router_kernel(x_ref, wg_ref, cw_ref, aux_ref, psum_ref, cnt1_ref):
    m = pl.program_id(0)
    nm = pl.num_programs(0)
    logits = jnp.dot(x_ref[...], wg_ref[...],
                     preferred_element_type=jnp.float32)
    mx = jnp.max(logits, axis=1, keepdims=True)
    ex = jnp.exp(logits - mx)
    probs = ex / jnp.sum(ex, axis=1, keepdims=True)
    eidx = jax.lax.broadcasted_iota(jnp.int32, probs.shape, 1)
    p1 = jnp.max(probs, axis=1, keepdims=True)
    i1 = jnp.min(jnp.where(probs == p1, eidx, E), axis=1, keepdims=True)
    probs2 = jnp.where(eidx == i1, -jnp.inf, probs)
    p2 = jnp.max(probs2, axis=1, keepdims=True)
    i2 = jnp.min(jnp.where(probs2 == p2, eidx, E), axis=1, keepdims=True)
    denom = p1 + p2 + 1e-9
    w1 = p1 / denom
    w2 = p2 / denom
    cw_ref[...] = jnp.where(eidx == i1, w1, 0.0) + jnp.where(eidx == i2, w2, 0.0)

    @pl.when(m == 0)
    def _init():
        psum_ref[...] = jnp.zeros_like(psum_ref)
        cnt1_ref[...] = jnp.zeros_like(cnt1_ref)

    psum_ref[...] += jnp.sum(probs, axis=0, keepdims=True)
    cnt1_ref[...] += jnp.sum((eidx == i1).astype(jnp.float32), axis=0,
                             keepdims=True)

    @pl.when(m == nm - 1)
    def _fin():
        f = cnt1_ref[...] / N
        pmean = psum_ref[...] / N
        aux_ref[...] = (E * jnp.sum(f * pmean)).reshape(1, 1)


def _ffn_kernel(x_ref, w1_ref, b1_ref, w2_ref, b2_ref, cw_ref, out_ref):
    e = pl.program_id(1)
    j = pl.program_id(2)
    h = jnp.dot(x_ref[...], w1_ref[0],
                preferred_element_type=jnp.float32) + b1_ref[0]
    h = _gelu(h)
    y = jnp.dot(h, w2_ref[0], preferred_element_type=jnp.float32)
    eidx = jax.lax.broadcasted_iota(jnp.int32, cw_ref.shape, 1)
    w = jnp.sum(jnp.where(eidx == e, cw_ref[...], 0.0), axis=1, keepdims=True)
    contrib = y * w

    @pl.when(j == 0)
    def _bias():
        contrib_b = contrib + b2_ref[0] * w

        @pl.when(e == 0)
        def _init():
            out_ref[...] = contrib_b

        @pl.when(e > 0)
        def _acc():
            out_ref[...] += contrib_b

    @pl.when(j > 0)
    def _accj():
        out_ref[...] += contrib


@jax.jit
def kernel(x, Wg, W1, b1, W2, b2):
    x_flat = x.reshape(N, D)
    TM = 512
    nm = N // TM

    cw, aux = pl.pallas_call(
        _router_kernel,
        grid=(nm,),
        in_specs=[
            pl.BlockSpec((TM, D), lambda m: (m, 0)),
            pl.BlockSpec((D, E), lambda m: (0, 0)),
        ],
        out_specs=[
            pl.BlockSpec((TM, E), lambda m: (m, 0)),
            pl.BlockSpec((1, 1), lambda m: (0, 0)),
            pl.BlockSpec((1, E), lambda m: (0, 0)),
            pl.BlockSpec((1, E), lambda m: (0, 0)),
        ],
        out_shape=[
            jax.ShapeDtypeStruct((N, E), jnp.float32),
            jax.ShapeDtypeStruct((1, 1), jnp.float32),
            jax.ShapeDtypeStruct((1, E), jnp.float32),
            jax.ShapeDtypeStruct((1, E), jnp.float32),
        ],
    )(x_flat, Wg)[:2]

    FFB = 1024
    nj = FF // FFB
    out = pl.pallas_call(
        _ffn_kernel,
        grid=(nm, E, nj),
        in_specs=[
            pl.BlockSpec((TM, D), lambda m, e, j: (m, 0)),
            pl.BlockSpec((1, D, FFB), lambda m, e, j: (e, 0, j)),
            pl.BlockSpec((1, 1, FFB), lambda m, e, j: (e, 0, j)),
            pl.BlockSpec((1, FFB, D), lambda m, e, j: (e, j, 0)),
            pl.BlockSpec((1, 1, D), lambda m, e, j: (e, 0, 0)),
            pl.BlockSpec((TM, E), lambda m, e, j: (m, 0)),
        ],
        out_specs=pl.BlockSpec((TM, D), lambda m, e, j: (m, 0)),
        out_shape=jax.ShapeDtypeStruct((N, D), jnp.float32),
    )(x_flat, W1, b1.reshape(E, 1, FF), W2, b2.reshape(E, 1, D), cw)

    return out.reshape(B, S, D), aux[0, 0]
